# Initial kernel scaffold; baseline (speedup 1.0000x reference)
import jax
import jax.numpy as jnp
from jax.experimental import pallas as pl


def kernel(x, gate_w, w1, b1, w2, b2):
    raise NotImplementedError("write your pallas kernel here")


# speedup vs baseline: 8.6179x; 8.6179x over previous
import functools

import jax
import jax.numpy as jnp
from jax.experimental import pallas as pl
from jax.experimental.pallas import tpu as pltpu

B, N, D, E, K = 2, 2048, 768, 8, 2
TB = 512


def _moe_block(x_ref, gw_ref, w1_ref, b1_ref, w2_ref, b2_ref, o_ref):
    xb = x_ref[...]
    logits = jnp.dot(xb, gw_ref[...], preferred_element_type=jnp.float32)
    probs = jax.nn.softmax(logits, axis=-1)
    e_ids = jax.lax.broadcasted_iota(jnp.int32, probs.shape, 1)
    i1 = jnp.argmax(probs, axis=-1)
    p1 = jnp.max(probs, axis=-1)
    sel1 = e_ids == i1[:, None]
    masked = jnp.where(sel1, -jnp.inf, probs)
    i2 = jnp.argmax(masked, axis=-1)
    p2 = jnp.max(masked, axis=-1)
    sel2 = e_ids == i2[:, None]
    wt = p1[:, None] * sel1.astype(jnp.float32) + p2[:, None] * sel2.astype(
        jnp.float32
    )

    acc = jnp.zeros((xb.shape[0], D), jnp.float32)
    inv_sqrt2 = 0.7071067811865476
    for e in range(E):
        h = jnp.dot(xb, w1_ref[e], preferred_element_type=jnp.float32)
        h = h + b1_ref[e][None, :]
        h = 0.5 * h * (1.0 + jax.lax.erf(h * inv_sqrt2))
        y = jnp.dot(h, w2_ref[e], preferred_element_type=jnp.float32)
        y = y + b2_ref[e][None, :]
        acc = acc + wt[:, e][:, None] * y
    o_ref[...] = acc


def kernel(x, gate_w, w1, b1, w2, b2):
    xf = x.reshape(B * N, D)
    grid = (B * N // TB,)
    out = pl.pallas_call(
        _moe_block,
        grid=grid,
        in_specs=[
            pl.BlockSpec((TB, D), lambda i: (i, 0)),
            pl.BlockSpec((D, E), lambda i: (0, 0)),
            pl.BlockSpec((E, D, D), lambda i: (0, 0, 0)),
            pl.BlockSpec((E, D), lambda i: (0, 0)),
            pl.BlockSpec((E, D, D), lambda i: (0, 0, 0)),
            pl.BlockSpec((E, D), lambda i: (0, 0)),
        ],
        out_specs=pl.BlockSpec((TB, D), lambda i: (i, 0)),
        out_shape=jax.ShapeDtypeStruct((B * N, D), jnp.float32),
        compiler_params=pltpu.CompilerParams(
            dimension_semantics=("arbitrary",),
        ),
    )(xf, gate_w, w1, b1, w2, b2)
    return out.reshape(B, N, D)
